# baseline (speedup 1.0000x reference)
import jax
import jax.numpy as jnp
from jax.experimental import pallas as pl
from jax.experimental.pallas import tpu as pltpu

_TILE = 512
_NT = 8


def _gin_kernel(adj_ref, x_ref, w1_ref, b1_ref, h_ref, pool_ref):
    g = pl.program_id(0)
    i = g % _NT

    agg = jnp.dot(adj_ref[0], x_ref[0], preferred_element_type=jnp.float32,
                  precision=jax.lax.Precision.DEFAULT)
    agg = agg + x_ref[0, pl.ds(i * _TILE, _TILE), :]

    h = jnp.dot(agg, w1_ref[...], preferred_element_type=jnp.float32,
                precision=jax.lax.Precision.DEFAULT)
    h = jnp.maximum(h + b1_ref[...], 0.0)
    h_ref[0] = h

    part = jnp.sum(h, axis=0, keepdims=True)[None]

    @pl.when(i == 0)
    def _():
        pool_ref[...] = part

    @pl.when(i != 0)
    def _():
        pool_ref[...] += part

    @pl.when(i == _NT - 1)
    def _():
        pool_ref[...] *= 1.0 / (_NT * _TILE)


def kernel(x, adj, W1, b1):
    B, N, D = x.shape
    n_tiles = N // _TILE
    b1_2d = b1.reshape(1, D)

    h, pooled = pl.pallas_call(
        _gin_kernel,
        grid=(B * n_tiles,),
        in_specs=[
            pl.BlockSpec((1, _TILE, N), lambda g: (g // _NT, g % _NT, 0)),
            pl.BlockSpec((1, N, D), lambda g: (g // _NT, 0, 0)),
            pl.BlockSpec((D, D), lambda g: (0, 0)),
            pl.BlockSpec((1, D), lambda g: (0, 0)),
        ],
        out_specs=[
            pl.BlockSpec((1, _TILE, D), lambda g: (g // _NT, g % _NT, 0)),
            pl.BlockSpec((1, 1, D), lambda g: (g // _NT, 0, 0)),
        ],
        out_shape=[
            jax.ShapeDtypeStruct((B, N, D), jnp.float32),
            jax.ShapeDtypeStruct((B, 1, D), jnp.float32),
        ],
        compiler_params=pltpu.CompilerParams(
            dimension_semantics=("arbitrary",),
        ),
    )(adj, x, W1, b1_2d)

    return (pooled.reshape(B, D), h)
